# baseline (speedup 1.0000x reference)
import jax
import jax.numpy as jnp
from jax.experimental import pallas as pl

B, C, D = 1, 2048, 768
H, P = 1536, 768
NS, NE, SHARED, K = 3, 8, 2, 2
LOCAL = NE - SHARED
T = NS * C
BTA = 512
NBLKA = T // BTA
BTB = 256
NBLKB = T // BTB
NBLKB_PER_SCALE = C // BTB
BTO = 256
NBLKO = C // BTO

_F32 = jnp.float32
_BF16 = jnp.bfloat16
_HI = jax.lax.Precision.HIGHEST
_SQRT_HALF = 0.7071067811865476


def _gelu(x):
    return 0.5 * x * (1.0 + jax.lax.erf(x * _SQRT_HALF))


def _rshared_body(x_ref, wr1_ref, br1_ref, wr2_ref, br2_ref,
                  wg1_ref, bg1_ref, wg2_ref, bg2_ref, wg3_ref, bg3_ref,
                  wloc_ref, gpred_ref, ent_ref):
    i = pl.program_id(0)
    x = x_ref[...]

    h = _gelu(jnp.dot(x, wr1_ref[...]) + br1_ref[...])
    logits = jnp.dot(h, wr2_ref[...]) + br2_ref[...]
    m = jnp.max(logits, axis=-1, keepdims=True)
    e = jnp.exp(logits - m)
    w = e / jnp.sum(e, axis=-1, keepdims=True)

    ent = jnp.sum(w * jnp.log(w + 1e-8))

    @pl.when(i == 0)
    def _():
        ent_ref[...] = jnp.zeros_like(ent_ref)
    ent_ref[...] = ent_ref[...] + ent

    lw = w[:, SHARED:]
    col = jax.lax.broadcasted_iota(jnp.int32, lw.shape, 1)
    m1 = jnp.max(lw, axis=-1, keepdims=True)
    i1 = jnp.min(jnp.where(lw == m1, col, LOCAL), axis=-1, keepdims=True)
    masked = jnp.where(col == i1, -1.0, lw)
    m2 = jnp.max(masked, axis=-1, keepdims=True)
    i2 = jnp.min(jnp.where(masked == m2, col, LOCAL), axis=-1, keepdims=True)
    wloc_ref[...] = (jnp.where(col == i1, m1, 0.0)
                     + jnp.where(col == i2, m2, 0.0))

    x16 = x.astype(_BF16)
    h1c = _gelu(jnp.dot(x16, wg1_ref[...], preferred_element_type=_F32)
                + bg1_ref[...])
    acc = jnp.zeros((BTA, P), _F32)
    for j in range(SHARED):
        h1 = h1c[:, j * H:(j + 1) * H].astype(_BF16)
        h2 = _gelu(jnp.dot(h1, wg2_ref[j], preferred_element_type=_F32)
                   + bg2_ref[j:j + 1])
        go = (jnp.dot(h2.astype(_BF16), wg3_ref[j],
                      preferred_element_type=_F32) + bg3_ref[j:j + 1])
        acc = acc + w[:, j:j + 1] * go
    gpred_ref[...] = acc.astype(_BF16)


def _local_body(x_ref, wloc_ref, gpred_ref,
                wl1_ref, bl1_ref, wl2_ref, bl2_ref,
                preds_ref, ssum_ref):
    i = pl.program_id(0)
    x16 = x_ref[...].astype(_BF16)
    wloc = wloc_ref[...]
    hc = _gelu(jnp.dot(x16, wl1_ref[...], preferred_element_type=_F32)
               + bl1_ref[...]).astype(_BF16)
    acc = gpred_ref[...].astype(_F32)
    for j in range(LOCAL):
        lo = (jnp.dot(hc[:, j * H:(j + 1) * H], wl2_ref[j],
                      preferred_element_type=_F32) + bl2_ref[j:j + 1])
        acc = acc + wloc[:, j:j + 1] * lo
    preds_ref[...] = acc.astype(_BF16)

    @pl.when(i % NBLKB_PER_SCALE == 0)
    def _():
        ssum_ref[...] = jnp.zeros_like(ssum_ref)
    ssum_ref[...] += jnp.sum(acc, axis=0, keepdims=True)[None]


def _scalew_body(ssum_ref, ent_ref, wt_ref, bt_ref, wmem_ref,
                 wwcr_ref, wwcm_ref, bwc_ref,
                 sw_ref, bal_ref):
    mean = ssum_ref[...] * (1.0 / C)
    r = _gelu(jnp.dot(mean, wt_ref[...], precision=_HI) + bt_ref[...])
    lg = bwc_ref[...] + jnp.dot(wmem_ref[...], wwcm_ref[...], precision=_HI)
    for s in range(NS):
        lg = lg + jnp.dot(r[s:s + 1], wwcr_ref[s], precision=_HI)
    raw = jax.nn.softplus(jnp.tanh(lg))
    m = jnp.max(raw, axis=-1, keepdims=True)
    e = jnp.exp(raw - m)
    sw_ref[...] = e / jnp.sum(e, axis=-1, keepdims=True)
    bal_ref[...] = 0.1 * (-(ent_ref[...] / T))


def _output_body(preds_ref, sw_ref, wo1_ref, bo1_ref, wo2_ref, bo2_ref,
                 out_ref):
    p = preds_ref[...].astype(_F32)
    sw = sw_ref[...]
    ws = sw[0, 0] * p[0] + sw[0, 1] * p[1] + sw[0, 2] * p[2]
    hh = _gelu(jnp.dot(ws.astype(_BF16), wo1_ref[...],
                       preferred_element_type=_F32) + bo1_ref[...])
    out_ref[...] = (jnp.dot(hh.astype(_BF16), wo2_ref[...],
                            preferred_element_type=_F32) + bo2_ref[...])


def _full(shape):
    return pl.BlockSpec(shape, lambda *_: tuple(0 for _ in shape))


@jax.jit
def kernel(x0, x1, x2, Wr1, br1, Wr2, br2, Wg1, bg1, Wg2, bg2, Wg3, bg3,
           Wl1, bl1, Wl2, bl2, Wt, bt, wmem, Wwc, bwc, Wo1, bo1, Wo2, bo2):
    xs = jnp.concatenate(
        [x0.reshape(C, D), x1.reshape(C, D), x2.reshape(C, D)], axis=0)

    Wg1c = jnp.transpose(Wg1, (1, 0, 2)).reshape(D, SHARED * H).astype(_BF16)
    bg1c = bg1.reshape(1, SHARED * H)
    Wl1c = jnp.transpose(Wl1, (1, 0, 2)).reshape(D, LOCAL * H).astype(_BF16)
    bl1c = bl1.reshape(1, LOCAL * H)

    wloc, gpred, ent = pl.pallas_call(
        _rshared_body,
        grid=(NBLKA,),
        in_specs=[
            pl.BlockSpec((BTA, D), lambda i: (i, 0)),
            _full((D, 2 * D)),
            _full((1, 2 * D)),
            _full((2 * D, NE)),
            _full((1, NE)),
            _full((D, SHARED * H)),
            _full((1, SHARED * H)),
            _full((SHARED, H, H)),
            _full((SHARED, H)),
            _full((SHARED, H, P)),
            _full((SHARED, P)),
        ],
        out_specs=[
            pl.BlockSpec((BTA, LOCAL), lambda i: (i, 0)),
            pl.BlockSpec((BTA, P), lambda i: (i, 0)),
            pl.BlockSpec((1, 1), lambda i: (0, 0)),
        ],
        out_shape=[
            jax.ShapeDtypeStruct((T, LOCAL), _F32),
            jax.ShapeDtypeStruct((T, P), _BF16),
            jax.ShapeDtypeStruct((1, 1), _F32),
        ],
    )(xs, Wr1, br1.reshape(1, -1), Wr2, br2.reshape(1, -1),
      Wg1c, bg1c, Wg2.astype(_BF16), bg2, Wg3.astype(_BF16), bg3)

    preds, ssum = pl.pallas_call(
        _local_body,
        grid=(NBLKB,),
        in_specs=[
            pl.BlockSpec((BTB, D), lambda i: (i, 0)),
            pl.BlockSpec((BTB, LOCAL), lambda i: (i, 0)),
            pl.BlockSpec((BTB, P), lambda i: (i, 0)),
            _full((D, LOCAL * H)),
            _full((1, LOCAL * H)),
            _full((LOCAL, H, P)),
            _full((LOCAL, P)),
        ],
        out_specs=[
            pl.BlockSpec((BTB, P), lambda i: (i, 0)),
            pl.BlockSpec((1, 1, P), lambda i: (i // NBLKB_PER_SCALE, 0, 0)),
        ],
        out_shape=[
            jax.ShapeDtypeStruct((T, P), _BF16),
            jax.ShapeDtypeStruct((NS, 1, P), _F32),
        ],
    )(xs, wloc, gpred, Wl1c, bl1c, Wl2.astype(_BF16), bl2)

    WwcR = Wwc[:NS * P].reshape(NS, P, NS)
    WwcM = Wwc[NS * P:]
    sw, bal = pl.pallas_call(
        _scalew_body,
        grid=(1,),
        in_specs=[
            _full((NS, P)),
            _full((1, 1)),
            _full((P, P)),
            _full((1, P)),
            _full((1, NS)),
            _full((NS, P, NS)),
            _full((NS, NS)),
            _full((1, NS)),
        ],
        out_specs=[
            pl.BlockSpec((1, NS), lambda i: (0, 0)),
            pl.BlockSpec((1, 1), lambda i: (0, 0)),
        ],
        out_shape=[
            jax.ShapeDtypeStruct((1, NS), _F32),
            jax.ShapeDtypeStruct((1, 1), _F32),
        ],
    )(ssum.reshape(NS, P), ent, Wt, bt.reshape(1, -1), wmem, WwcR, WwcM,
      bwc.reshape(1, -1))

    out = pl.pallas_call(
        _output_body,
        grid=(NBLKO,),
        in_specs=[
            pl.BlockSpec((NS, BTO, P), lambda i: (0, i, 0)),
            _full((1, NS)),
            _full((P, P)),
            _full((1, P)),
            _full((P, P)),
            _full((1, P)),
        ],
        out_specs=pl.BlockSpec((BTO, P), lambda i: (i, 0)),
        out_shape=jax.ShapeDtypeStruct((C, P), _F32),
    )(preds.reshape(NS, C, P), sw,
      Wo1.astype(_BF16), bo1.reshape(1, -1),
      Wo2.astype(_BF16), bo2.reshape(1, -1))

    return out.reshape(B, C, P), bal.reshape(())


# speedup vs baseline: 1.0033x; 1.0033x over previous
import jax
import jax.numpy as jnp
from jax.experimental import pallas as pl

B, C, D = 1, 2048, 768
H, P = 1536, 768
NS, NE, SHARED, K = 3, 8, 2, 2
LOCAL = NE - SHARED
T = NS * C
BT = 256
NBLK = T // BT
NBLK_PER_SCALE = C // BT
BTO = 256
NBLKO = C // BTO

_F32 = jnp.float32
_BF16 = jnp.bfloat16
_HI = jax.lax.Precision.HIGHEST
_SQRT_HALF = 0.7071067811865476


def _gelu(x):
    return 0.5 * x * (1.0 + jax.lax.erf(x * _SQRT_HALF))


def _moe_body(x_ref, wr1_ref, br1_ref, wr2_ref, br2_ref,
              wg1_ref, bg1_ref, wg2_ref, bg2_ref, wg3_ref, bg3_ref,
              wl1_ref, bl1_ref, wl2_ref, bl2_ref,
              preds_ref, ssum_ref, ent_ref):
    i = pl.program_id(0)
    x = x_ref[...]

    h = _gelu(jnp.dot(x, wr1_ref[...]) + br1_ref[...])
    logits = jnp.dot(h, wr2_ref[...]) + br2_ref[...]
    m = jnp.max(logits, axis=-1, keepdims=True)
    e = jnp.exp(logits - m)
    w = e / jnp.sum(e, axis=-1, keepdims=True)

    ent = jnp.sum(w * jnp.log(w + 1e-8))

    @pl.when(i == 0)
    def _():
        ent_ref[...] = jnp.zeros_like(ent_ref)
    ent_ref[...] = ent_ref[...] + ent

    lw = w[:, SHARED:]
    col = jax.lax.broadcasted_iota(jnp.int32, lw.shape, 1)
    m1 = jnp.max(lw, axis=-1, keepdims=True)
    i1 = jnp.min(jnp.where(lw == m1, col, LOCAL), axis=-1, keepdims=True)
    masked = jnp.where(col == i1, -1.0, lw)
    m2 = jnp.max(masked, axis=-1, keepdims=True)
    i2 = jnp.min(jnp.where(masked == m2, col, LOCAL), axis=-1, keepdims=True)
    wloc = (jnp.where(col == i1, m1, 0.0)
            + jnp.where(col == i2, m2, 0.0))

    x16 = x.astype(_BF16)
    h1c = _gelu(jnp.dot(x16, wg1_ref[...], preferred_element_type=_F32)
                + bg1_ref[...])
    acc = jnp.zeros((BT, P), _F32)
    for j in range(SHARED):
        h1 = h1c[:, j * H:(j + 1) * H].astype(_BF16)
        h2 = _gelu(jnp.dot(h1, wg2_ref[j], preferred_element_type=_F32)
                   + bg2_ref[j:j + 1])
        go = (jnp.dot(h2.astype(_BF16), wg3_ref[j],
                      preferred_element_type=_F32) + bg3_ref[j:j + 1])
        acc = acc + w[:, j:j + 1] * go
    for k in range(LOCAL // 2):
        hp = _gelu(jnp.dot(x16, wl1_ref[:, 2 * k * H:2 * (k + 1) * H],
                           preferred_element_type=_F32)
                   + bl1_ref[:, 2 * k * H:2 * (k + 1) * H]).astype(_BF16)
        for jj in range(2):
            j = 2 * k + jj
            lo = (jnp.dot(hp[:, jj * H:(jj + 1) * H], wl2_ref[j],
                          preferred_element_type=_F32) + bl2_ref[j:j + 1])
            acc = acc + wloc[:, j:j + 1] * lo
    preds_ref[...] = acc.astype(_BF16)

    @pl.when(i % NBLK_PER_SCALE == 0)
    def _():
        ssum_ref[...] = jnp.zeros_like(ssum_ref)
    ssum_ref[...] += jnp.sum(acc, axis=0, keepdims=True)[None]


def _scalew_body(ssum_ref, ent_ref, wt_ref, bt_ref, wmem_ref,
                 wwcr_ref, wwcm_ref, bwc_ref,
                 sw_ref, bal_ref):
    mean = ssum_ref[...] * (1.0 / C)
    r = _gelu(jnp.dot(mean, wt_ref[...], precision=_HI) + bt_ref[...])
    lg = bwc_ref[...] + jnp.dot(wmem_ref[...], wwcm_ref[...], precision=_HI)
    for s in range(NS):
        lg = lg + jnp.dot(r[s:s + 1], wwcr_ref[s], precision=_HI)
    raw = jax.nn.softplus(jnp.tanh(lg))
    m = jnp.max(raw, axis=-1, keepdims=True)
    e = jnp.exp(raw - m)
    sw_ref[...] = e / jnp.sum(e, axis=-1, keepdims=True)
    bal_ref[...] = 0.1 * (-(ent_ref[...] / T))


def _output_body(preds_ref, sw_ref, wo1_ref, bo1_ref, wo2_ref, bo2_ref,
                 out_ref):
    p = preds_ref[...].astype(_F32)
    sw = sw_ref[...]
    ws = sw[0, 0] * p[0] + sw[0, 1] * p[1] + sw[0, 2] * p[2]
    hh = _gelu(jnp.dot(ws.astype(_BF16), wo1_ref[...],
                       preferred_element_type=_F32) + bo1_ref[...])
    out_ref[...] = (jnp.dot(hh.astype(_BF16), wo2_ref[...],
                            preferred_element_type=_F32) + bo2_ref[...])


def _full(shape):
    return pl.BlockSpec(shape, lambda *_: tuple(0 for _ in shape))


@jax.jit
def kernel(x0, x1, x2, Wr1, br1, Wr2, br2, Wg1, bg1, Wg2, bg2, Wg3, bg3,
           Wl1, bl1, Wl2, bl2, Wt, bt, wmem, Wwc, bwc, Wo1, bo1, Wo2, bo2):
    xs = jnp.concatenate(
        [x0.reshape(C, D), x1.reshape(C, D), x2.reshape(C, D)], axis=0)

    Wg1c = jnp.transpose(Wg1, (1, 0, 2)).reshape(D, SHARED * H).astype(_BF16)
    bg1c = bg1.reshape(1, SHARED * H)
    Wl1c = jnp.transpose(Wl1, (1, 0, 2)).reshape(D, LOCAL * H).astype(_BF16)
    bl1c = bl1.reshape(1, LOCAL * H)

    preds, ssum, ent = pl.pallas_call(
        _moe_body,
        grid=(NBLK,),
        in_specs=[
            pl.BlockSpec((BT, D), lambda i: (i, 0)),
            _full((D, 2 * D)),
            _full((1, 2 * D)),
            _full((2 * D, NE)),
            _full((1, NE)),
            _full((D, SHARED * H)),
            _full((1, SHARED * H)),
            _full((SHARED, H, H)),
            _full((SHARED, H)),
            _full((SHARED, H, P)),
            _full((SHARED, P)),
            _full((D, LOCAL * H)),
            _full((1, LOCAL * H)),
            _full((LOCAL, H, P)),
            _full((LOCAL, P)),
        ],
        out_specs=[
            pl.BlockSpec((BT, P), lambda i: (i, 0)),
            pl.BlockSpec((1, 1, P), lambda i: (i // NBLK_PER_SCALE, 0, 0)),
            pl.BlockSpec((1, 1), lambda i: (0, 0)),
        ],
        out_shape=[
            jax.ShapeDtypeStruct((T, P), _BF16),
            jax.ShapeDtypeStruct((NS, 1, P), _F32),
            jax.ShapeDtypeStruct((1, 1), _F32),
        ],
    )(xs, Wr1, br1.reshape(1, -1), Wr2, br2.reshape(1, -1),
      Wg1c, bg1c, Wg2.astype(_BF16), bg2, Wg3.astype(_BF16), bg3,
      Wl1c, bl1c, Wl2.astype(_BF16), bl2)

    WwcR = Wwc[:NS * P].reshape(NS, P, NS)
    WwcM = Wwc[NS * P:]
    sw, bal = pl.pallas_call(
        _scalew_body,
        grid=(1,),
        in_specs=[
            _full((NS, P)),
            _full((1, 1)),
            _full((P, P)),
            _full((1, P)),
            _full((1, NS)),
            _full((NS, P, NS)),
            _full((NS, NS)),
            _full((1, NS)),
        ],
        out_specs=[
            pl.BlockSpec((1, NS), lambda i: (0, 0)),
            pl.BlockSpec((1, 1), lambda i: (0, 0)),
        ],
        out_shape=[
            jax.ShapeDtypeStruct((1, NS), _F32),
            jax.ShapeDtypeStruct((1, 1), _F32),
        ],
    )(ssum.reshape(NS, P), ent, Wt, bt.reshape(1, -1), wmem, WwcR, WwcM,
      bwc.reshape(1, -1))

    out = pl.pallas_call(
        _output_body,
        grid=(NBLKO,),
        in_specs=[
            pl.BlockSpec((NS, BTO, P), lambda i: (0, i, 0)),
            _full((1, NS)),
            _full((P, P)),
            _full((1, P)),
            _full((P, P)),
            _full((1, P)),
        ],
        out_specs=pl.BlockSpec((BTO, P), lambda i: (i, 0)),
        out_shape=jax.ShapeDtypeStruct((C, P), _F32),
    )(preds.reshape(NS, C, P), sw,
      Wo1.astype(_BF16), bo1.reshape(1, -1),
      Wo2.astype(_BF16), bo2.reshape(1, -1))

    return out.reshape(B, C, P), bal.reshape(())


# speedup vs baseline: 1.0913x; 1.0878x over previous
import jax
import jax.numpy as jnp
from jax.experimental import pallas as pl

B, C, D = 1, 2048, 768
H, P = 1536, 768
NS, NE, SHARED, K = 3, 8, 2, 2
LOCAL = NE - SHARED
T = NS * C
BT = 256
NBLK = T // BT
NBLK_PER_SCALE = C // BT

_F32 = jnp.float32
_BF16 = jnp.bfloat16
_HI = jax.lax.Precision.HIGHEST
_SQRT_HALF = 0.7071067811865476


def _gelu(x):
    return 0.5 * x * (1.0 + jax.lax.erf(x * _SQRT_HALF))


def _moe_body(x_ref, wr1_ref, br1_ref, wr2_ref, br2_ref,
              wg1_ref, bg1_ref, wg2_ref, bg2_ref, wg3_ref, bg3_ref,
              wl1_ref, bl1_ref, wl2_ref, bl2_ref,
              preds_ref, ssum_ref, ent_ref):
    i = pl.program_id(0)
    x = x_ref[...]

    h = _gelu(jnp.dot(x, wr1_ref[...]) + br1_ref[...])
    logits = jnp.dot(h, wr2_ref[...]) + br2_ref[...]
    m = jnp.max(logits, axis=-1, keepdims=True)
    e = jnp.exp(logits - m)
    w = e / jnp.sum(e, axis=-1, keepdims=True)

    ent = jnp.sum(w * jnp.log(w + 1e-8))

    @pl.when(i == 0)
    def _():
        ent_ref[...] = jnp.zeros_like(ent_ref)
    ent_ref[...] = ent_ref[...] + ent

    lw = w[:, SHARED:]
    col = jax.lax.broadcasted_iota(jnp.int32, lw.shape, 1)
    m1 = jnp.max(lw, axis=-1, keepdims=True)
    i1 = jnp.min(jnp.where(lw == m1, col, LOCAL), axis=-1, keepdims=True)
    masked = jnp.where(col == i1, -1.0, lw)
    m2 = jnp.max(masked, axis=-1, keepdims=True)
    i2 = jnp.min(jnp.where(masked == m2, col, LOCAL), axis=-1, keepdims=True)
    wloc = (jnp.where(col == i1, m1, 0.0)
            + jnp.where(col == i2, m2, 0.0))

    x16 = x.astype(_BF16)
    acc = jnp.zeros((BT, P), _F32)
    for j in range(SHARED):
        h1 = _gelu(jnp.dot(x16, wg1_ref[j], preferred_element_type=_F32)
                   + bg1_ref[j:j + 1])
        h2 = _gelu(jnp.dot(h1.astype(_BF16), wg2_ref[j],
                           preferred_element_type=_F32) + bg2_ref[j:j + 1])
        go = (jnp.dot(h2.astype(_BF16), wg3_ref[j],
                      preferred_element_type=_F32) + bg3_ref[j:j + 1])
        acc = acc + w[:, j:j + 1] * go
    for j in range(LOCAL):
        hl = _gelu(jnp.dot(x16, wl1_ref[j], preferred_element_type=_F32)
                   + bl1_ref[j:j + 1])
        lo = (jnp.dot(hl.astype(_BF16), wl2_ref[j],
                      preferred_element_type=_F32) + bl2_ref[j:j + 1])
        acc = acc + wloc[:, j:j + 1] * lo
    preds_ref[...] = acc.astype(_BF16)

    @pl.when(i % NBLK_PER_SCALE == 0)
    def _():
        ssum_ref[...] = jnp.zeros_like(ssum_ref)
    ssum_ref[...] += jnp.sum(acc, axis=0, keepdims=True)[None]


def _scalew_body(ssum_ref, ent_ref, wt_ref, bt_ref, wmem_ref,
                 wwcr_ref, wwcm_ref, bwc_ref,
                 sw_ref, bal_ref):
    mean = ssum_ref[...] * (1.0 / C)
    r = _gelu(jnp.dot(mean, wt_ref[...], precision=_HI) + bt_ref[...])
    lg = bwc_ref[...] + jnp.dot(wmem_ref[...], wwcm_ref[...], precision=_HI)
    for s in range(NS):
        lg = lg + jnp.dot(r[s:s + 1], wwcr_ref[s], precision=_HI)
    raw = jax.nn.softplus(jnp.tanh(lg))
    m = jnp.max(raw, axis=-1, keepdims=True)
    e = jnp.exp(raw - m)
    sw_ref[...] = e / jnp.sum(e, axis=-1, keepdims=True)
    bal_ref[...] = 0.1 * (-(ent_ref[...] / T))


def _output_body(preds_ref, sw_ref, wo1_ref, bo1_ref, wo2_ref, bo2_ref,
                 out_ref):
    p = preds_ref[...].astype(_F32)
    sw = sw_ref[...]
    ws = sw[0, 0] * p[0] + sw[0, 1] * p[1] + sw[0, 2] * p[2]
    hh = _gelu(jnp.dot(ws.astype(_BF16), wo1_ref[...],
                       preferred_element_type=_F32) + bo1_ref[...])
    out_ref[...] = (jnp.dot(hh.astype(_BF16), wo2_ref[...],
                            preferred_element_type=_F32) + bo2_ref[...])


def _full(shape):
    return pl.BlockSpec(shape, lambda *_: tuple(0 for _ in shape))


@jax.jit
def kernel(x0, x1, x2, Wr1, br1, Wr2, br2, Wg1, bg1, Wg2, bg2, Wg3, bg3,
           Wl1, bl1, Wl2, bl2, Wt, bt, wmem, Wwc, bwc, Wo1, bo1, Wo2, bo2):
    xs = jnp.concatenate(
        [x0.reshape(C, D), x1.reshape(C, D), x2.reshape(C, D)], axis=0)

    preds, ssum, ent = pl.pallas_call(
        _moe_body,
        grid=(NBLK,),
        in_specs=[
            pl.BlockSpec((BT, D), lambda i: (i, 0)),
            _full((D, 2 * D)),
            _full((1, 2 * D)),
            _full((2 * D, NE)),
            _full((1, NE)),
            _full((SHARED, D, H)),
            _full((SHARED, H)),
            _full((SHARED, H, H)),
            _full((SHARED, H)),
            _full((SHARED, H, P)),
            _full((SHARED, P)),
            _full((LOCAL, D, H)),
            _full((LOCAL, H)),
            _full((LOCAL, H, P)),
            _full((LOCAL, P)),
        ],
        out_specs=[
            pl.BlockSpec((BT, P), lambda i: (i, 0)),
            pl.BlockSpec((1, 1, P), lambda i: (i // NBLK_PER_SCALE, 0, 0)),
            pl.BlockSpec((1, 1), lambda i: (0, 0)),
        ],
        out_shape=[
            jax.ShapeDtypeStruct((T, P), _BF16),
            jax.ShapeDtypeStruct((NS, 1, P), _F32),
            jax.ShapeDtypeStruct((1, 1), _F32),
        ],
    )(xs, Wr1, br1.reshape(1, -1), Wr2, br2.reshape(1, -1),
      Wg1.astype(_BF16), bg1, Wg2.astype(_BF16), bg2,
      Wg3.astype(_BF16), bg3,
      Wl1.astype(_BF16), bl1, Wl2.astype(_BF16), bl2)

    WwcR = Wwc[:NS * P].reshape(NS, P, NS)
    WwcM = Wwc[NS * P:]
    sw, bal = pl.pallas_call(
        _scalew_body,
        grid=(1,),
        in_specs=[
            _full((NS, P)),
            _full((1, 1)),
            _full((P, P)),
            _full((1, P)),
            _full((1, NS)),
            _full((NS, P, NS)),
            _full((NS, NS)),
            _full((1, NS)),
        ],
        out_specs=[
            pl.BlockSpec((1, NS), lambda i: (0, 0)),
            pl.BlockSpec((1, 1), lambda i: (0, 0)),
        ],
        out_shape=[
            jax.ShapeDtypeStruct((1, NS), _F32),
            jax.ShapeDtypeStruct((1, 1), _F32),
        ],
    )(ssum.reshape(NS, P), ent, Wt, bt.reshape(1, -1), wmem, WwcR, WwcM,
      bwc.reshape(1, -1))

    out = pl.pallas_call(
        _output_body,
        grid=(NBLK_PER_SCALE,),
        in_specs=[
            pl.BlockSpec((NS, BT, P), lambda i: (0, i, 0)),
            _full((1, NS)),
            _full((P, P)),
            _full((1, P)),
            _full((P, P)),
            _full((1, P)),
        ],
        out_specs=pl.BlockSpec((BT, P), lambda i: (i, 0)),
        out_shape=jax.ShapeDtypeStruct((C, P), _F32),
    )(preds.reshape(NS, C, P), sw,
      Wo1.astype(_BF16), bo1.reshape(1, -1),
      Wo2.astype(_BF16), bo2.reshape(1, -1))

    return out.reshape(B, C, P), bal.reshape(())


# speedup vs baseline: 1.1016x; 1.0094x over previous
import jax
import jax.numpy as jnp
from jax.experimental import pallas as pl

B, C, D = 1, 2048, 768
H, P = 1536, 768
NS, NE, SHARED, K = 3, 8, 2, 2
LOCAL = NE - SHARED
T = NS * C
BT = 512
NBLK = T // BT
NBLK_PER_SCALE = C // BT

_F32 = jnp.float32
_BF16 = jnp.bfloat16
_HI = jax.lax.Precision.HIGHEST
_SQRT_HALF = 0.7071067811865476


def _gelu(x):
    return 0.5 * x * (1.0 + jax.lax.erf(x * _SQRT_HALF))


def _moe_body(x_ref, wr1_ref, br1_ref, wr2_ref, br2_ref,
              wg1_ref, bg1_ref, wg2_ref, bg2_ref, wg3_ref, bg3_ref,
              wl1_ref, bl1_ref, wl2_ref, bl2_ref,
              preds_ref, ssum_ref, ent_ref):
    i = pl.program_id(0)
    x16 = x_ref[...]

    h = _gelu(jnp.dot(x16, wr1_ref[...], preferred_element_type=_F32)
              + br1_ref[...])
    logits = jnp.dot(h, wr2_ref[...]) + br2_ref[...]
    m = jnp.max(logits, axis=-1, keepdims=True)
    e = jnp.exp(logits - m)
    w = e / jnp.sum(e, axis=-1, keepdims=True)

    ent = jnp.sum(w * jnp.log(w + 1e-8))

    @pl.when(i == 0)
    def _():
        ent_ref[...] = jnp.zeros_like(ent_ref)
    ent_ref[...] = ent_ref[...] + ent

    lw = w[:, SHARED:]
    col = jax.lax.broadcasted_iota(jnp.int32, lw.shape, 1)
    m1 = jnp.max(lw, axis=-1, keepdims=True)
    i1 = jnp.min(jnp.where(lw == m1, col, LOCAL), axis=-1, keepdims=True)
    masked = jnp.where(col == i1, -1.0, lw)
    m2 = jnp.max(masked, axis=-1, keepdims=True)
    i2 = jnp.min(jnp.where(masked == m2, col, LOCAL), axis=-1, keepdims=True)
    wloc = (jnp.where(col == i1, m1, 0.0)
            + jnp.where(col == i2, m2, 0.0))

    acc = jnp.zeros((BT, P), _F32)
    for j in range(SHARED):
        h1 = _gelu(jnp.dot(x16, wg1_ref[j], preferred_element_type=_F32)
                   + bg1_ref[j:j + 1])
        h2 = _gelu(jnp.dot(h1.astype(_BF16), wg2_ref[j],
                           preferred_element_type=_F32) + bg2_ref[j:j + 1])
        go = (jnp.dot(h2.astype(_BF16), wg3_ref[j],
                      preferred_element_type=_F32) + bg3_ref[j:j + 1])
        acc = acc + w[:, j:j + 1] * go
    for j in range(LOCAL):
        hl = _gelu(jnp.dot(x16, wl1_ref[j], preferred_element_type=_F32)
                   + bl1_ref[j:j + 1])
        lo = (jnp.dot(hl.astype(_BF16), wl2_ref[j],
                      preferred_element_type=_F32) + bl2_ref[j:j + 1])
        acc = acc + wloc[:, j:j + 1] * lo
    preds_ref[...] = acc.astype(_BF16)

    @pl.when(i % NBLK_PER_SCALE == 0)
    def _():
        ssum_ref[...] = jnp.zeros_like(ssum_ref)
    ssum_ref[...] += jnp.sum(acc, axis=0, keepdims=True)[None]


def _scalew_body(ssum_ref, ent_ref, wt_ref, bt_ref, wmem_ref,
                 wwcr_ref, wwcm_ref, bwc_ref,
                 sw_ref, bal_ref):
    mean = ssum_ref[...] * (1.0 / C)
    r = _gelu(jnp.dot(mean, wt_ref[...], precision=_HI) + bt_ref[...])
    lg = bwc_ref[...] + jnp.dot(wmem_ref[...], wwcm_ref[...], precision=_HI)
    for s in range(NS):
        lg = lg + jnp.dot(r[s:s + 1], wwcr_ref[s], precision=_HI)
    raw = jax.nn.softplus(jnp.tanh(lg))
    m = jnp.max(raw, axis=-1, keepdims=True)
    e = jnp.exp(raw - m)
    sw_ref[...] = e / jnp.sum(e, axis=-1, keepdims=True)
    bal_ref[...] = 0.1 * (-(ent_ref[...] / T))


def _output_body(preds_ref, sw_ref, wo1_ref, bo1_ref, wo2_ref, bo2_ref,
                 out_ref):
    p = preds_ref[...].astype(_F32)
    sw = sw_ref[...]
    ws = sw[0, 0] * p[0] + sw[0, 1] * p[1] + sw[0, 2] * p[2]
    hh = _gelu(jnp.dot(ws.astype(_BF16), wo1_ref[...],
                       preferred_element_type=_F32) + bo1_ref[...])
    out_ref[...] = (jnp.dot(hh.astype(_BF16), wo2_ref[...],
                            preferred_element_type=_F32) + bo2_ref[...])


def _full(shape):
    return pl.BlockSpec(shape, lambda *_: tuple(0 for _ in shape))


@jax.jit
def kernel(x0, x1, x2, Wr1, br1, Wr2, br2, Wg1, bg1, Wg2, bg2, Wg3, bg3,
           Wl1, bl1, Wl2, bl2, Wt, bt, wmem, Wwc, bwc, Wo1, bo1, Wo2, bo2):
    xs = jnp.concatenate(
        [x0.reshape(C, D), x1.reshape(C, D), x2.reshape(C, D)], axis=0)

    preds, ssum, ent = pl.pallas_call(
        _moe_body,
        grid=(NBLK,),
        in_specs=[
            pl.BlockSpec((BT, D), lambda i: (i, 0)),
            _full((D, 2 * D)),
            _full((1, 2 * D)),
            _full((2 * D, NE)),
            _full((1, NE)),
            _full((SHARED, D, H)),
            _full((SHARED, H)),
            _full((SHARED, H, H)),
            _full((SHARED, H)),
            _full((SHARED, H, P)),
            _full((SHARED, P)),
            _full((LOCAL, D, H)),
            _full((LOCAL, H)),
            _full((LOCAL, H, P)),
            _full((LOCAL, P)),
        ],
        out_specs=[
            pl.BlockSpec((BT, P), lambda i: (i, 0)),
            pl.BlockSpec((1, 1, P), lambda i: (i // NBLK_PER_SCALE, 0, 0)),
            pl.BlockSpec((1, 1), lambda i: (0, 0)),
        ],
        out_shape=[
            jax.ShapeDtypeStruct((T, P), _BF16),
            jax.ShapeDtypeStruct((NS, 1, P), _F32),
            jax.ShapeDtypeStruct((1, 1), _F32),
        ],
    )(xs.astype(_BF16), Wr1.astype(_BF16), br1.reshape(1, -1),
      Wr2, br2.reshape(1, -1),
      Wg1.astype(_BF16), bg1, Wg2.astype(_BF16), bg2,
      Wg3.astype(_BF16), bg3,
      Wl1.astype(_BF16), bl1, Wl2.astype(_BF16), bl2)

    WwcR = Wwc[:NS * P].reshape(NS, P, NS)
    WwcM = Wwc[NS * P:]
    sw, bal = pl.pallas_call(
        _scalew_body,
        grid=(1,),
        in_specs=[
            _full((NS, P)),
            _full((1, 1)),
            _full((P, P)),
            _full((1, P)),
            _full((1, NS)),
            _full((NS, P, NS)),
            _full((NS, NS)),
            _full((1, NS)),
        ],
        out_specs=[
            pl.BlockSpec((1, NS), lambda i: (0, 0)),
            pl.BlockSpec((1, 1), lambda i: (0, 0)),
        ],
        out_shape=[
            jax.ShapeDtypeStruct((1, NS), _F32),
            jax.ShapeDtypeStruct((1, 1), _F32),
        ],
    )(ssum.reshape(NS, P), ent, Wt, bt.reshape(1, -1), wmem, WwcR, WwcM,
      bwc.reshape(1, -1))

    out = pl.pallas_call(
        _output_body,
        grid=(NBLK_PER_SCALE,),
        in_specs=[
            pl.BlockSpec((NS, BT, P), lambda i: (0, i, 0)),
            _full((1, NS)),
            _full((P, P)),
            _full((1, P)),
            _full((P, P)),
            _full((1, P)),
        ],
        out_specs=pl.BlockSpec((BT, P), lambda i: (i, 0)),
        out_shape=jax.ShapeDtypeStruct((C, P), _F32),
    )(preds.reshape(NS, C, P), sw,
      Wo1.astype(_BF16), bo1.reshape(1, -1),
      Wo2.astype(_BF16), bo2.reshape(1, -1))

    return out.reshape(B, C, P), bal.reshape(())


# speedup vs baseline: 1.1047x; 1.0028x over previous
import jax
import jax.numpy as jnp
from jax.experimental import pallas as pl
from jax.experimental.pallas import tpu as pltpu

B, C, D = 1, 2048, 768
H, P = 1536, 768
NS, NE, SHARED, K = 3, 8, 2, 2
LOCAL = NE - SHARED
T = NS * C
BT = 512
NBLK = T // BT
NBLK_PER_SCALE = C // BT

_F32 = jnp.float32
_BF16 = jnp.bfloat16
_HI = jax.lax.Precision.HIGHEST
_SQRT_HALF = 0.7071067811865476


def _gelu(x):
    return 0.5 * x * (1.0 + jax.lax.erf(x * _SQRT_HALF))


def _moe_body(x_ref, wr1_ref, br1_ref, wr2_ref, br2_ref,
              wg1_ref, bg1_ref, wg2_ref, bg2_ref, wg3_ref, bg3_ref,
              wl1_ref, bl1_ref, wl2_ref, bl2_ref,
              preds_ref, ssum_ref, ent_ref):
    i = pl.program_id(0)
    x16 = x_ref[...]

    h = _gelu(jnp.dot(x16, wr1_ref[...], preferred_element_type=_F32)
              + br1_ref[...])
    logits = jnp.dot(h, wr2_ref[...]) + br2_ref[...]
    m = jnp.max(logits, axis=-1, keepdims=True)
    e = jnp.exp(logits - m)
    w = e / jnp.sum(e, axis=-1, keepdims=True)

    ent = jnp.sum(w * jnp.log(w + 1e-8))

    @pl.when(i == 0)
    def _():
        ent_ref[...] = jnp.zeros_like(ent_ref)
    ent_ref[...] = ent_ref[...] + ent

    lw = w[:, SHARED:]
    col = jax.lax.broadcasted_iota(jnp.int32, lw.shape, 1)
    m1 = jnp.max(lw, axis=-1, keepdims=True)
    i1 = jnp.min(jnp.where(lw == m1, col, LOCAL), axis=-1, keepdims=True)
    masked = jnp.where(col == i1, -1.0, lw)
    m2 = jnp.max(masked, axis=-1, keepdims=True)
    i2 = jnp.min(jnp.where(masked == m2, col, LOCAL), axis=-1, keepdims=True)
    wloc = (jnp.where(col == i1, m1, 0.0)
            + jnp.where(col == i2, m2, 0.0))

    acc = jnp.zeros((BT, P), _F32)
    for j in range(SHARED):
        h1 = _gelu(jnp.dot(x16, wg1_ref[j], preferred_element_type=_F32)
                   + bg1_ref[j:j + 1])
        h2 = _gelu(jnp.dot(h1.astype(_BF16), wg2_ref[j],
                           preferred_element_type=_F32) + bg2_ref[j:j + 1])
        go = (jnp.dot(h2.astype(_BF16), wg3_ref[j],
                      preferred_element_type=_F32) + bg3_ref[j:j + 1])
        acc = acc + w[:, j:j + 1] * go
    for j in range(LOCAL):
        hl = _gelu(jnp.dot(x16, wl1_ref[j], preferred_element_type=_F32)
                   + bl1_ref[j:j + 1])
        lo = (jnp.dot(hl.astype(_BF16), wl2_ref[j],
                      preferred_element_type=_F32) + bl2_ref[j:j + 1])
        acc = acc + wloc[:, j:j + 1] * lo
    preds_ref[...] = acc.astype(_BF16)

    @pl.when(i % NBLK_PER_SCALE == 0)
    def _():
        ssum_ref[...] = jnp.zeros_like(ssum_ref)
    ssum_ref[...] += jnp.sum(acc, axis=0, keepdims=True)[None]


def _outsw_body(ssum_ref, ent_ref, wt_ref, bt_ref, wmem_ref,
                wwcr_ref, wwcm_ref, bwc_ref,
                preds_ref, wo1_ref, bo1_ref, wo2_ref, bo2_ref,
                out_ref, bal_ref, sw_ref):
    i = pl.program_id(0)

    @pl.when(i == 0)
    def _():
        mean = ssum_ref[...] * (1.0 / C)
        r = _gelu(jnp.dot(mean, wt_ref[...], precision=_HI) + bt_ref[...])
        lg = (bwc_ref[...]
              + jnp.dot(wmem_ref[...], wwcm_ref[...], precision=_HI))
        for s in range(NS):
            lg = lg + jnp.dot(r[s:s + 1], wwcr_ref[s], precision=_HI)
        raw = jax.nn.softplus(jnp.tanh(lg))
        m = jnp.max(raw, axis=-1, keepdims=True)
        e = jnp.exp(raw - m)
        sw_ref[...] = e / jnp.sum(e, axis=-1, keepdims=True)
        bal_ref[...] = 0.1 * (-(ent_ref[...] / T))

    p = preds_ref[...].astype(_F32)
    sw = sw_ref[...]
    ws = sw[0, 0] * p[0] + sw[0, 1] * p[1] + sw[0, 2] * p[2]
    hh = _gelu(jnp.dot(ws.astype(_BF16), wo1_ref[...],
                       preferred_element_type=_F32) + bo1_ref[...])
    out_ref[...] = (jnp.dot(hh.astype(_BF16), wo2_ref[...],
                            preferred_element_type=_F32) + bo2_ref[...])


def _full(shape):
    return pl.BlockSpec(shape, lambda *_: tuple(0 for _ in shape))


@jax.jit
def kernel(x0, x1, x2, Wr1, br1, Wr2, br2, Wg1, bg1, Wg2, bg2, Wg3, bg3,
           Wl1, bl1, Wl2, bl2, Wt, bt, wmem, Wwc, bwc, Wo1, bo1, Wo2, bo2):
    xs = jnp.concatenate(
        [x0.reshape(C, D), x1.reshape(C, D), x2.reshape(C, D)], axis=0)

    preds, ssum, ent = pl.pallas_call(
        _moe_body,
        grid=(NBLK,),
        in_specs=[
            pl.BlockSpec((BT, D), lambda i: (i, 0)),
            _full((D, 2 * D)),
            _full((1, 2 * D)),
            _full((2 * D, NE)),
            _full((1, NE)),
            _full((SHARED, D, H)),
            _full((SHARED, H)),
            _full((SHARED, H, H)),
            _full((SHARED, H)),
            _full((SHARED, H, P)),
            _full((SHARED, P)),
            _full((LOCAL, D, H)),
            _full((LOCAL, H)),
            _full((LOCAL, H, P)),
            _full((LOCAL, P)),
        ],
        out_specs=[
            pl.BlockSpec((BT, P), lambda i: (i, 0)),
            pl.BlockSpec((1, 1, P), lambda i: (i // NBLK_PER_SCALE, 0, 0)),
            pl.BlockSpec((1, 1), lambda i: (0, 0)),
        ],
        out_shape=[
            jax.ShapeDtypeStruct((T, P), _BF16),
            jax.ShapeDtypeStruct((NS, 1, P), _F32),
            jax.ShapeDtypeStruct((1, 1), _F32),
        ],
    )(xs.astype(_BF16), Wr1.astype(_BF16), br1.reshape(1, -1),
      Wr2, br2.reshape(1, -1),
      Wg1.astype(_BF16), bg1, Wg2.astype(_BF16), bg2,
      Wg3.astype(_BF16), bg3,
      Wl1.astype(_BF16), bl1, Wl2.astype(_BF16), bl2)

    WwcR = Wwc[:NS * P].reshape(NS, P, NS)
    WwcM = Wwc[NS * P:]
    out, bal = pl.pallas_call(
        _outsw_body,
        grid=(NBLK_PER_SCALE,),
        in_specs=[
            _full((NS, P)),
            _full((1, 1)),
            _full((P, P)),
            _full((1, P)),
            _full((1, NS)),
            _full((NS, P, NS)),
            _full((NS, NS)),
            _full((1, NS)),
            pl.BlockSpec((NS, BT, P), lambda i: (0, i, 0)),
            _full((P, P)),
            _full((1, P)),
            _full((P, P)),
            _full((1, P)),
        ],
        out_specs=[
            pl.BlockSpec((BT, P), lambda i: (i, 0)),
            pl.BlockSpec((1, 1), lambda i: (0, 0)),
        ],
        out_shape=[
            jax.ShapeDtypeStruct((C, P), _F32),
            jax.ShapeDtypeStruct((1, 1), _F32),
        ],
        scratch_shapes=[pltpu.VMEM((1, NS), _F32)],
    )(ssum.reshape(NS, P), ent, Wt, bt.reshape(1, -1), wmem, WwcR, WwcM,
      bwc.reshape(1, -1), preds.reshape(NS, C, P),
      Wo1.astype(_BF16), bo1.reshape(1, -1),
      Wo2.astype(_BF16), bo2.reshape(1, -1))

    return out.reshape(B, C, P), bal.reshape(())
